# Initial kernel scaffold; baseline (speedup 1.0000x reference)
import jax
import jax.numpy as jnp
from jax.experimental import pallas as pl


def kernel(x, table):
    raise NotImplementedError("write your pallas kernel here")


# speedup vs baseline: 3.3918x; 3.3918x over previous
import functools

import jax
import jax.numpy as jnp
from jax import lax
from jax.experimental import pallas as pl
from jax.experimental.pallas import tpu as pltpu
from jax.experimental.pallas import tpu_sc as plsc

_N_FIELDS = 26
_EMBED_DIM = 64
_BATCH = 4096
_TOTAL = _BATCH * _N_FIELDS
_NC, _NS, _LANES = 2, 16, 16
_NW = _NC * _NS
_PER_W = _TOTAL // _NW
_ROW = 128
_G = _PER_W // _ROW


_mesh = plsc.VectorSubcoreMesh(core_axis_name="c", subcore_axis_name="s")


@functools.partial(
    pl.kernel,
    mesh=_mesh,
    out_type=jax.ShapeDtypeStruct((_TOTAL, _EMBED_DIM), jnp.float32),
    scratch_types=[
        pltpu.VMEM((_G, _ROW), jnp.int32),
        pltpu.VMEM((_ROW, _EMBED_DIM), jnp.float32),
        pltpu.SemaphoreType.DMA,
    ],
    compiler_params=pltpu.CompilerParams(use_tc_tiling_on_sc=False),
)
def _emb_lookup(x_hbm, table_hbm, out_hbm, idx_v, rows_v, sem):
    wid = lax.axis_index("s") * _NC + lax.axis_index("c")
    base_blk = wid * _G

    lane = lax.iota(jnp.int32, _LANES)
    pats = {
        s: ((s + lane) % _N_FIELDS) * 1000 for s in range(0, _N_FIELDS, 2)
    }

    pltpu.sync_copy(x_hbm.at[wid], idx_v)
    for g in range(_G):
        row = idx_v.at[g]
        for j in range(_ROW // _LANES):
            s = pl.ds(j * _LANES, _LANES)
            row[s] = row[s] + pats[(g * _ROW + j * _LANES) % _N_FIELDS]

    for g in range(_G):
        pltpu.async_copy(table_hbm.at[idx_v.at[g]], rows_v, sem).wait()
        pltpu.sync_copy(rows_v, out_hbm.at[pl.ds((base_blk + g) * _ROW, _ROW)])


def kernel(x, table):
    out = _emb_lookup(x.reshape(_NW, _G, _ROW), table)
    return out.reshape(_BATCH, _N_FIELDS, _EMBED_DIM)


# speedup vs baseline: 3.8414x; 1.1326x over previous
import functools

import jax
import jax.numpy as jnp
from jax import lax
from jax.experimental import pallas as pl
from jax.experimental.pallas import tpu as pltpu
from jax.experimental.pallas import tpu_sc as plsc

_N_FIELDS = 26
_EMBED_DIM = 64
_BATCH = 4096
_TOTAL = _BATCH * _N_FIELDS
_NC, _NS, _LANES = 2, 16, 16
_NW = _NC * _NS
_PER_W = _TOTAL // _NW
_ROW = 128
_G = _PER_W // _ROW


_mesh = plsc.VectorSubcoreMesh(core_axis_name="c", subcore_axis_name="s")


_NB = 4


@functools.partial(
    pl.kernel,
    mesh=_mesh,
    out_type=jax.ShapeDtypeStruct((_TOTAL, _EMBED_DIM), jnp.float32),
    scratch_types=[
        pltpu.VMEM((_G, _ROW), jnp.int32),
        pltpu.VMEM((_NB, _ROW, _EMBED_DIM), jnp.float32),
        [pltpu.SemaphoreType.DMA] * _NB,
        [pltpu.SemaphoreType.DMA] * _NB,
    ],
    compiler_params=pltpu.CompilerParams(use_tc_tiling_on_sc=False),
)
def _emb_lookup(x_hbm, table_hbm, out_hbm, idx_v, rows_v, gsems, wsems):
    wid = lax.axis_index("s") * _NC + lax.axis_index("c")
    base_blk = wid * _G

    lane = lax.iota(jnp.int32, _LANES)
    pats = {
        s: ((s + lane) % _N_FIELDS) * 1000 for s in range(0, _N_FIELDS, 2)
    }

    pltpu.sync_copy(x_hbm.at[wid], idx_v)

    def fix(g):
        row = idx_v.at[g]
        for j in range(_ROW // _LANES):
            s = pl.ds(j * _LANES, _LANES)
            row[s] = row[s] + pats[(g * _ROW + j * _LANES) % _N_FIELDS]

    def gather(g, b):
        return pltpu.async_copy(table_hbm.at[idx_v.at[g]], rows_v.at[b],
                                gsems[b])

    def write(g, b):
        return pltpu.async_copy(
            rows_v.at[b], out_hbm.at[pl.ds((base_blk + g) * _ROW, _ROW)],
            wsems[b])

    hg = [None] * _NB
    hw = [None] * _NB
    for b in range(_NB):
        fix(b)
        hg[b] = gather(b, b)
    for g in range(_G):
        b = g % _NB
        hg[b].wait()
        hw[b] = write(g, b)
        nxt = g + _NB
        if nxt < _G:
            fix(nxt)
            hw[b].wait()
            hg[b] = gather(nxt, b)
    for g in range(max(0, _G - _NB), _G):
        hw[g % _NB].wait()


def kernel(x, table):
    out = _emb_lookup(x.reshape(_NW, _G, _ROW), table)
    return out.reshape(_BATCH, _N_FIELDS, _EMBED_DIM)
